# baseline (speedup 1.0000x reference)
import functools

import jax
import jax.numpy as jnp
from jax import lax
from jax.experimental import pallas as pl
from jax.experimental.pallas import tpu as pltpu
from jax.experimental.pallas import tpu_sc as plsc

N_FACTORS = 32
BATCH = 16384
NUM_CORES = 2
NUM_SUBCORES = 16
NUM_WORKERS = NUM_CORES * NUM_SUBCORES
LANES = 16
B_PER_W = BATCH // NUM_WORKERS
CHUNK = 128
CHUNKS = B_PER_W // CHUNK

_mesh = plsc.VectorSubcoreMesh(core_axis_name="c", subcore_axis_name="s")


@functools.partial(
    pl.kernel,
    mesh=_mesh,
    out_type=jax.ShapeDtypeStruct((BATCH,), jnp.float32),
    compiler_params=pltpu.CompilerParams(
        use_tc_tiling_on_sc=False, needs_layout_passes=False),
    scratch_types=[
        pltpu.VMEM((CHUNKS, CHUNK), jnp.int32),
        pltpu.VMEM((CHUNKS, CHUNK), jnp.int32),
        pltpu.VMEM((B_PER_W, N_FACTORS), jnp.float32),
        pltpu.VMEM((B_PER_W, N_FACTORS), jnp.float32),
        pltpu.VMEM((B_PER_W,), jnp.float32),
        pltpu.SemaphoreType.DMA,
    ],
)
def _mf_sc(user_hbm, item_hbm, uf_hbm, vf_hbm, out_hbm,
           uidx, iidx, urows, vrows, outv, sem):
    wid = lax.axis_index("s") * NUM_CORES + lax.axis_index("c")
    base = wid * B_PER_W

    pltpu.sync_copy(user_hbm.at[pl.ds(wid * CHUNKS, CHUNKS)], uidx)
    pltpu.sync_copy(item_hbm.at[pl.ds(wid * CHUNKS, CHUNKS)], iidx)

    copies = []
    for c in range(CHUNKS):
        copies.append(pltpu.async_copy(
            uf_hbm.at[uidx.at[c]], urows.at[pl.ds(c * CHUNK, CHUNK)], sem))
        copies.append(pltpu.async_copy(
            vf_hbm.at[iidx.at[c]], vrows.at[pl.ds(c * CHUNK, CHUNK)], sem))
    for cp in copies:
        cp.wait()

    lanes = lax.iota(jnp.int32, LANES)

    def body(g, carry):
        rows = g * LANES + lanes
        acc = jnp.zeros((LANES,), jnp.float32)
        for f in range(N_FACTORS):
            col = jnp.full((LANES,), f, jnp.int32)
            uu = plsc.load_gather(urows, [rows, col])
            vv = plsc.load_gather(vrows, [rows, col])
            acc = acc + uu * vv
        outv[pl.ds(pl.multiple_of(g * LANES, LANES), LANES)] = acc
        return carry

    lax.fori_loop(0, B_PER_W // LANES, body, 0)

    pltpu.sync_copy(outv, out_hbm.at[pl.ds(base, B_PER_W)])


def kernel(user, item, user_factors, item_factors):
    u2 = user.reshape(NUM_WORKERS * CHUNKS, CHUNK)
    i2 = item.reshape(NUM_WORKERS * CHUNKS, CHUNK)
    return _mf_sc(u2, i2, user_factors, item_factors)
